# baseline (speedup 1.0000x reference)
import functools
import math

import jax
import jax.numpy as jnp
from jax.experimental import pallas as pl
from jax.experimental.pallas import tpu as pltpu

_TILE = 16
_MASK_VAL = -10000000000.0


def _blockdiag_lanes(a, heads, head_dim):
    d_model, k = a.shape
    out = jnp.concatenate([a] * heads, axis=1)
    row_h = jax.lax.broadcasted_iota(jnp.int32, (d_model, heads * k), 0) // head_dim
    col_h = jax.lax.broadcasted_iota(jnp.int32, (d_model, heads * k), 1) // k
    return jnp.where(row_h == col_h, out, 0.0)


def _blockdiag_sublanes(a, heads, head_dim):
    k, d_model = a.shape
    out = jnp.concatenate([a] * heads, axis=0)
    row_h = jax.lax.broadcasted_iota(jnp.int32, (heads * k, d_model), 0) // k
    col_h = jax.lax.broadcasted_iota(jnp.int32, (heads * k, d_model), 1) // head_dim
    return jnp.where(row_h == col_h, out, 0.0)


def _kascade_kernel(xf_ref, xl_ref, x_ref, wq_ref, wk_ref, wv_ref, wo_ref,
                    out_ref, qk_s, vo_s, qk16_s, vo16_s, gg_s, gg16_s,
                    *, tile_q, seq_len, heads, head_dim):
    t = pl.program_id(1)
    num_t = seq_len // tile_q
    half = heads * _TILE
    width = 2 * half
    last_tile_start = ((seq_len - 1) // _TILE) * _TILE
    scale = 1.0 / math.sqrt(head_dim)

    @pl.when(t == 0)
    def _prep():
        xs = jnp.concatenate([xf_ref[0], xl_ref[0]], axis=0)
        kst = jax.lax.dot_general(wk_ref[...], xs, (((0,), (1,)), ((), ())),
                                  preferred_element_type=jnp.float32)
        kbd_a = _blockdiag_lanes(kst[:, :_TILE], heads, head_dim)
        kbd_b = _blockdiag_lanes(kst[:, _TILE:], heads, head_dim)
        qk_a = jnp.dot(wq_ref[...], kbd_a,
                       preferred_element_type=jnp.float32) * scale
        qk_b = jnp.dot(wq_ref[...], kbd_b,
                       preferred_element_type=jnp.float32) * scale
        qk_s[...] = jnp.concatenate([qk_a, qk_b], axis=1)
        qk16_s[...] = qk_a

        vsf = jnp.dot(xs, wv_ref[...], preferred_element_type=jnp.float32)
        vbd_a = _blockdiag_sublanes(vsf[:_TILE], heads, head_dim)
        vbd_b = _blockdiag_sublanes(vsf[_TILE:], heads, head_dim)
        vo_a = jnp.dot(vbd_a, wo_ref[...], preferred_element_type=jnp.float32)
        vo_b = jnp.dot(vbd_b, wo_ref[...], preferred_element_type=jnp.float32)
        vo_s[...] = jnp.concatenate([vo_a, vo_b], axis=0)
        vo16_s[...] = vo_a

        rw = jax.lax.broadcasted_iota(jnp.int32, (width, width), 0)
        cw = jax.lax.broadcasted_iota(jnp.int32, (width, width), 1)
        gg_s[...] = jnp.where((rw % half) // _TILE == (cw % half) // _TILE,
                              1.0, 0.0)
        rh = jax.lax.broadcasted_iota(jnp.int32, (half, half), 0)
        ch = jax.lax.broadcasted_iota(jnp.int32, (half, half), 1)
        gg16_s[...] = jnp.where(rh // _TILE == ch // _TILE, 1.0, 0.0)

    def _stream(apply_causal):
        l = jnp.dot(x_ref[0], qk16_s[...], preferred_element_type=jnp.float32)
        if apply_causal:
            q_idx = jax.lax.broadcasted_iota(jnp.int32, (tile_q, half), 0)
            k_tok = jax.lax.broadcasted_iota(jnp.int32, (tile_q, half), 1) % _TILE
            l = jnp.where(k_tok > q_idx, _MASK_VAL, l)
        m = l[:, :1]
        e = jnp.exp(l - m)
        d = jnp.dot(e, gg16_s[...], preferred_element_type=jnp.float32)
        out_ref[0] = jnp.dot(e / d, vo16_s[...],
                             preferred_element_type=jnp.float32)

    @pl.when(t == 0)
    def _stream_first():
        _stream(True)

    @pl.when(t > 0)
    def _stream_rest():
        _stream(False)

    @pl.when(t == num_t - 1)
    def _fixup():
        x16 = x_ref[0, tile_q - _TILE:, :]
        l = jnp.dot(x16, qk_s[...], preferred_element_type=jnp.float32)
        q_idx = seq_len - _TILE + jax.lax.broadcasted_iota(
            jnp.int32, (_TILE, width), 0)
        j = jax.lax.broadcasted_iota(jnp.int32, (_TILE, width), 1)
        k_tok = jnp.where(j < half, j % _TILE, last_tile_start + j % _TILE)
        l = jnp.where(k_tok > q_idx, _MASK_VAL, l)
        m = jnp.max(l, axis=-1, keepdims=True)
        e = jnp.exp(l - m)
        d = jnp.dot(e, gg_s[...], preferred_element_type=jnp.float32)
        out_ref[0, tile_q - _TILE:, :] = jnp.dot(
            e / d, vo_s[...], preferred_element_type=jnp.float32)


@jax.jit
def kernel(x, Wq, Wk, Wv, Wo):
    batch, seq_len, d_model = x.shape
    head_dim = 64
    heads = Wq.shape[1] // head_dim
    tile_q = 2048
    last_tile_start = ((seq_len - 1) // _TILE) * _TILE
    half = heads * _TILE
    width = 2 * half

    out = pl.pallas_call(
        functools.partial(_kascade_kernel, tile_q=tile_q, seq_len=seq_len,
                          heads=heads, head_dim=head_dim),
        grid=(batch, seq_len // tile_q),
        in_specs=[
            pl.BlockSpec((1, _TILE, d_model), lambda b, t: (b, 0, 0)),
            pl.BlockSpec((1, _TILE, d_model),
                         lambda b, t: (b, last_tile_start // _TILE, 0)),
            pl.BlockSpec((1, tile_q, d_model), lambda b, t: (b, t, 0)),
            pl.BlockSpec((d_model, heads * head_dim), lambda b, t: (0, 0)),
            pl.BlockSpec((d_model, heads * head_dim), lambda b, t: (0, 0)),
            pl.BlockSpec((d_model, heads * head_dim), lambda b, t: (0, 0)),
            pl.BlockSpec((heads * head_dim, d_model), lambda b, t: (0, 0)),
        ],
        out_specs=pl.BlockSpec((1, tile_q, d_model), lambda b, t: (b, t, 0)),
        out_shape=jax.ShapeDtypeStruct((batch, seq_len, d_model), jnp.float32),
        compiler_params=pltpu.CompilerParams(
            dimension_semantics=("parallel", "arbitrary")),
        scratch_shapes=[
            pltpu.VMEM((d_model, width), jnp.float32),
            pltpu.VMEM((width, d_model), jnp.float32),
            pltpu.VMEM((d_model, half), jnp.float32),
            pltpu.VMEM((half, d_model), jnp.float32),
            pltpu.VMEM((width, width), jnp.float32),
            pltpu.VMEM((half, half), jnp.float32),
        ],
    )(x, x, x, Wq, Wk, Wv, Wo)
    return out


# speedup vs baseline: 1.0035x; 1.0035x over previous
import functools
import math

import jax
import jax.numpy as jnp
from jax.experimental import pallas as pl
from jax.experimental.pallas import tpu as pltpu

_TILE = 16
_MASK_VAL = -10000000000.0


def _blockdiag_lanes(a, heads, head_dim):
    d_model, k = a.shape
    out = jnp.concatenate([a] * heads, axis=1)
    row_h = jax.lax.broadcasted_iota(jnp.int32, (d_model, heads * k), 0) // head_dim
    col_h = jax.lax.broadcasted_iota(jnp.int32, (d_model, heads * k), 1) // k
    return jnp.where(row_h == col_h, out, 0.0)


def _blockdiag_sublanes(a, heads, head_dim):
    k, d_model = a.shape
    out = jnp.concatenate([a] * heads, axis=0)
    row_h = jax.lax.broadcasted_iota(jnp.int32, (heads * k, d_model), 0) // k
    col_h = jax.lax.broadcasted_iota(jnp.int32, (heads * k, d_model), 1) // head_dim
    return jnp.where(row_h == col_h, out, 0.0)


def _kascade_kernel(xf_ref, xl_ref, x_ref, wq_ref, wk_ref, wv_ref, wo_ref,
                    out_ref, qk_s, vo_s, qk16_s, vo16_s, gg_s, gg16_s,
                    *, tile_q, seq_len, heads, head_dim):
    t = pl.program_id(1)
    num_t = seq_len // tile_q
    half = heads * _TILE
    width = 2 * half
    last_tile_start = ((seq_len - 1) // _TILE) * _TILE
    scale = 1.0 / math.sqrt(head_dim)

    @pl.when(t == 0)
    def _prep():
        xs = jnp.concatenate([xf_ref[0], xl_ref[0]], axis=0)
        kst = jax.lax.dot_general(wk_ref[...], xs, (((0,), (1,)), ((), ())),
                                  preferred_element_type=jnp.float32)
        kbd_a = _blockdiag_lanes(kst[:, :_TILE], heads, head_dim)
        kbd_b = _blockdiag_lanes(kst[:, _TILE:], heads, head_dim)
        qk_a = jnp.dot(wq_ref[...], kbd_a,
                       preferred_element_type=jnp.float32) * scale
        qk_b = jnp.dot(wq_ref[...], kbd_b,
                       preferred_element_type=jnp.float32) * scale
        qk_s[...] = jnp.concatenate([qk_a, qk_b], axis=1)
        qk16_s[...] = qk_a

        vsf = jnp.dot(xs, wv_ref[...], preferred_element_type=jnp.float32)
        vbd_a = _blockdiag_sublanes(vsf[:_TILE], heads, head_dim)
        vbd_b = _blockdiag_sublanes(vsf[_TILE:], heads, head_dim)
        vo_a = jnp.dot(vbd_a, wo_ref[...], preferred_element_type=jnp.float32)
        vo_b = jnp.dot(vbd_b, wo_ref[...], preferred_element_type=jnp.float32)
        vo_s[...] = jnp.concatenate([vo_a, vo_b], axis=0)
        vo16_s[...] = vo_a

        rw = jax.lax.broadcasted_iota(jnp.int32, (width, width), 0)
        cw = jax.lax.broadcasted_iota(jnp.int32, (width, width), 1)
        gg_s[...] = jnp.where((rw % half) // _TILE == (cw % half) // _TILE,
                              1.0, 0.0)
        rh = jax.lax.broadcasted_iota(jnp.int32, (half, half), 0)
        ch = jax.lax.broadcasted_iota(jnp.int32, (half, half), 1)
        gg16_s[...] = jnp.where(rh // _TILE == ch // _TILE, 1.0, 0.0)

    def _stream(apply_causal):
        l = jnp.dot(x_ref[0], qk16_s[...], preferred_element_type=jnp.float32)
        if apply_causal:
            q_idx = jax.lax.broadcasted_iota(jnp.int32, (tile_q, half), 0)
            k_tok = jax.lax.broadcasted_iota(jnp.int32, (tile_q, half), 1) % _TILE
            l = jnp.where(k_tok > q_idx, _MASK_VAL, l)
        m = jnp.max(l, axis=-1, keepdims=True)
        e = jnp.exp(l - m)
        d = jnp.dot(e, gg16_s[...], preferred_element_type=jnp.float32)
        out_ref[0] = jnp.dot(e / d, vo16_s[...],
                             preferred_element_type=jnp.float32)

    @pl.when(t == 0)
    def _stream_first():
        _stream(True)

    @pl.when(t > 0)
    def _stream_rest():
        _stream(False)

    @pl.when(t == num_t - 1)
    def _fixup():
        x16 = x_ref[0, tile_q - _TILE:, :]
        l = jnp.dot(x16, qk_s[...], preferred_element_type=jnp.float32)
        q_idx = seq_len - _TILE + jax.lax.broadcasted_iota(
            jnp.int32, (_TILE, width), 0)
        j = jax.lax.broadcasted_iota(jnp.int32, (_TILE, width), 1)
        k_tok = jnp.where(j < half, j % _TILE, last_tile_start + j % _TILE)
        l = jnp.where(k_tok > q_idx, _MASK_VAL, l)
        m = jnp.max(l, axis=-1, keepdims=True)
        e = jnp.exp(l - m)
        d = jnp.dot(e, gg_s[...], preferred_element_type=jnp.float32)
        out_ref[0, tile_q - _TILE:, :] = jnp.dot(
            e / d, vo_s[...], preferred_element_type=jnp.float32)


@jax.jit
def kernel(x, Wq, Wk, Wv, Wo):
    batch, seq_len, d_model = x.shape
    head_dim = 64
    heads = Wq.shape[1] // head_dim
    tile_q = 2048
    last_tile_start = ((seq_len - 1) // _TILE) * _TILE
    half = heads * _TILE
    width = 2 * half

    out = pl.pallas_call(
        functools.partial(_kascade_kernel, tile_q=tile_q, seq_len=seq_len,
                          heads=heads, head_dim=head_dim),
        grid=(batch, seq_len // tile_q),
        in_specs=[
            pl.BlockSpec((1, _TILE, d_model), lambda b, t: (b, 0, 0)),
            pl.BlockSpec((1, _TILE, d_model),
                         lambda b, t: (b, last_tile_start // _TILE, 0)),
            pl.BlockSpec((1, tile_q, d_model), lambda b, t: (b, t, 0)),
            pl.BlockSpec((d_model, heads * head_dim), lambda b, t: (0, 0)),
            pl.BlockSpec((d_model, heads * head_dim), lambda b, t: (0, 0)),
            pl.BlockSpec((d_model, heads * head_dim), lambda b, t: (0, 0)),
            pl.BlockSpec((heads * head_dim, d_model), lambda b, t: (0, 0)),
        ],
        out_specs=pl.BlockSpec((1, tile_q, d_model), lambda b, t: (b, t, 0)),
        out_shape=jax.ShapeDtypeStruct((batch, seq_len, d_model), jnp.float32),
        compiler_params=pltpu.CompilerParams(
            dimension_semantics=("parallel", "arbitrary")),
        scratch_shapes=[
            pltpu.VMEM((d_model, width), jnp.float32),
            pltpu.VMEM((width, d_model), jnp.float32),
            pltpu.VMEM((d_model, half), jnp.float32),
            pltpu.VMEM((half, d_model), jnp.float32),
            pltpu.VMEM((width, width), jnp.float32),
            pltpu.VMEM((half, half), jnp.float32),
        ],
    )(x, x, x, Wq, Wk, Wv, Wo)
    return out
